# baseline (speedup 1.0000x reference)
import jax
import jax.numpy as jnp
from jax import lax
from jax.experimental import pallas as pl
from jax.experimental.pallas import tpu as pltpu

_ROWS = 32768
_COLS = 2048
_CH = 256
_NCHUNK = _ROWS // _CH
_DEPTH = 12


def _loss_kernel(x_hbm, y_hbm, out_ref, xb, yb, sx, sy):
    def start(i):
        slot = lax.rem(i, _DEPTH)
        pltpu.make_async_copy(
            x_hbm.at[pl.ds(i * _CH, _CH)], xb.at[slot], sx.at[slot]).start()
        pltpu.make_async_copy(
            y_hbm.at[pl.ds(i * _CH, _CH)], yb.at[slot], sy.at[slot]).start()

    for i in range(_DEPTH):
        start(i)

    def step(i, accs):
        a_d, a_y = accs
        slot = lax.rem(i, _DEPTH)
        pltpu.make_async_copy(
            x_hbm.at[pl.ds(i * _CH, _CH)], xb.at[slot], sx.at[slot]).wait()
        pltpu.make_async_copy(
            y_hbm.at[pl.ds(i * _CH, _CH)], yb.at[slot], sy.at[slot]).wait()
        x = xb[slot]
        y = yb[slot]
        d = x - y
        a_d = a_d + jnp.sum(d * d)
        a_y = a_y + jnp.sum(y * y)

        @pl.when(i + _DEPTH < _NCHUNK)
        def _():
            start(i + _DEPTH)

        return (a_d, a_y)

    a_d, a_y = lax.fori_loop(0, _NCHUNK, step, (0.0, 0.0))
    out_ref[0] = a_d / (2.0 * a_y)


def kernel(x, y):
    x2 = x.reshape(_ROWS, _COLS)
    y2 = y.reshape(_ROWS, _COLS)
    out = pl.pallas_call(
        _loss_kernel,
        in_specs=[
            pl.BlockSpec(memory_space=pl.ANY),
            pl.BlockSpec(memory_space=pl.ANY),
        ],
        out_specs=pl.BlockSpec(memory_space=pltpu.SMEM),
        out_shape=jax.ShapeDtypeStruct((1,), jnp.float32),
        scratch_shapes=[
            pltpu.VMEM((_DEPTH, _CH, _COLS), jnp.float32),
            pltpu.VMEM((_DEPTH, _CH, _COLS), jnp.float32),
            pltpu.SemaphoreType.DMA((_DEPTH,)),
            pltpu.SemaphoreType.DMA((_DEPTH,)),
        ],
    )(x2, y2)
    return out[0]
